# Initial kernel scaffold; baseline (speedup 1.0000x reference)
import jax
import jax.numpy as jnp
from jax.experimental import pallas as pl


def kernel(positions, weight, paths):
    raise NotImplementedError("write your pallas kernel here")


# speedup vs baseline: 1.0533x; 1.0533x over previous
import functools

import jax
import jax.numpy as jnp
from jax.experimental import pallas as pl
from jax.experimental.pallas import tpu as pltpu


def _expand_body(pos_ref, w_ref, out_ref):
    C = out_ref.shape[1]
    D = w_ref.shape[1]
    max_depth = C // (2 * D)

    c = jax.lax.broadcasted_iota(jnp.int32, (1, C), 1)
    k = c // D
    t = k // 2
    par = k % 2
    e = k % max_depth

    w = jnp.tanh(w_ref[...])
    scale = jnp.sqrt((1.0 - w * w) * (D / 2.0))
    wt = jnp.concatenate([w] * (2 * max_depth), axis=1)
    st = jnp.concatenate([scale] * (2 * max_depth), axis=1)
    w2 = wt * wt
    w4 = w2 * w2
    w8 = w4 * w4
    v = st
    v = v * jnp.where((e & 1) != 0, wt, 1.0)
    v = v * jnp.where((e & 2) != 0, w2, 1.0)
    v = v * jnp.where((e & 4) != 0, w4, 1.0)
    v = v * jnp.where((e & 8) != 0, w8, 1.0)

    m = jnp.maximum(pos_ref[...], 1)
    sh = jnp.right_shift(m, t)
    bit = (sh >= 2) & ((sh & 1) == par)
    out_ref[...] = jnp.where(bit, v, 0.0)


@functools.partial(jax.jit, static_argnames=("rows_per_block", "word_len"))
def _expand(pos_flat, weight_row, rows_per_block=1024, word_len=20):
    n_rows = pos_flat.shape[0]
    d = weight_row.shape[1]
    c = word_len * d
    grid = (n_rows // rows_per_block,)
    return pl.pallas_call(
        _expand_body,
        grid=grid,
        in_specs=[
            pl.BlockSpec((rows_per_block, 1), lambda i: (i, 0)),
            pl.BlockSpec((1, d), lambda i: (0, 0)),
        ],
        out_specs=pl.BlockSpec((rows_per_block, c), lambda i: (i, 0)),
        out_shape=jax.ShapeDtypeStruct((n_rows, c), jnp.float32),
        compiler_params=pltpu.CompilerParams(
            dimension_semantics=("arbitrary",),
        ),
    )(pos_flat, weight_row)


def kernel(positions, weight, paths):
    b, s = positions.shape
    d = weight.shape[0]
    word_len = paths.shape[1]
    pos_flat = positions.reshape(b * s, 1)
    out = _expand(pos_flat, weight.reshape(1, d), word_len=word_len)
    return out.reshape(b, s, word_len * d)


# speedup vs baseline: 2.4544x; 2.3302x over previous
import functools

import jax
import jax.numpy as jnp
from jax.experimental import pallas as pl
from jax.experimental.pallas import tpu as pltpu


def _expand_body(pos_ref, w_ref, out_ref):
    C = out_ref.shape[2]
    D = w_ref.shape[1]
    max_depth = C // (2 * D)

    c = jax.lax.broadcasted_iota(jnp.int32, (1, 1, C), 2)
    k = c // D
    t = k // 2
    par = k % 2
    e = k % max_depth

    w = jnp.tanh(w_ref[...])
    scale = jnp.sqrt((1.0 - w * w) * (D / 2.0))
    wt = jnp.concatenate([w] * (2 * max_depth), axis=1)
    st = jnp.concatenate([scale] * (2 * max_depth), axis=1)
    w2 = wt * wt
    w4 = w2 * w2
    w8 = w4 * w4
    e2 = e[0]
    v = st
    v = v * jnp.where((e2 & 1) != 0, wt, 1.0)
    v = v * jnp.where((e2 & 2) != 0, w2, 1.0)
    v = v * jnp.where((e2 & 4) != 0, w4, 1.0)
    v = v * jnp.where((e2 & 8) != 0, w8, 1.0)
    v = v[None]

    m = jnp.maximum(pos_ref[...], 1)
    sh = jnp.right_shift(m[:, :, None], t)
    bit = (sh >= 2) & ((sh & 1) == par)
    out_ref[...] = jnp.where(bit, v, 0.0)


@functools.partial(jax.jit, static_argnames=("block_b", "word_len"))
def _expand(positions, weight_row, block_b=32, word_len=20):
    b, s = positions.shape
    d = weight_row.shape[1]
    c = word_len * d
    grid = (b // block_b,)
    return pl.pallas_call(
        _expand_body,
        grid=grid,
        in_specs=[
            pl.BlockSpec((block_b, s), lambda i: (i, 0)),
            pl.BlockSpec((1, d), lambda i: (0, 0)),
        ],
        out_specs=pl.BlockSpec((block_b, s, c), lambda i: (i, 0, 0)),
        out_shape=jax.ShapeDtypeStruct((b, s, c), jnp.float32),
        compiler_params=pltpu.CompilerParams(
            dimension_semantics=("arbitrary",),
        ),
    )(positions, weight_row)


def kernel(positions, weight, paths):
    d = weight.shape[0]
    word_len = paths.shape[1]
    return _expand(positions, weight.reshape(1, d), word_len=word_len)


# speedup vs baseline: 2.5201x; 1.0268x over previous
import functools

import jax
import jax.numpy as jnp
from jax.experimental import pallas as pl
from jax.experimental.pallas import tpu as pltpu


def _expand_body(pos_ref, w_ref, out_ref):
    C = out_ref.shape[2]
    D = w_ref.shape[1]
    max_depth = C // (2 * D)

    c = jax.lax.broadcasted_iota(jnp.int32, (1, 1, C), 2)
    k = c // D
    t = k // 2
    par = k % 2
    e = k % max_depth

    w = jnp.tanh(w_ref[...])
    scale = jnp.sqrt((1.0 - w * w) * (D / 2.0))
    wt = jnp.concatenate([w] * (2 * max_depth), axis=1)
    st = jnp.concatenate([scale] * (2 * max_depth), axis=1)
    w2 = wt * wt
    w4 = w2 * w2
    w8 = w4 * w4
    e2 = e[0]
    v = st
    v = v * jnp.where((e2 & 1) != 0, wt, 1.0)
    v = v * jnp.where((e2 & 2) != 0, w2, 1.0)
    v = v * jnp.where((e2 & 4) != 0, w4, 1.0)
    v = v * jnp.where((e2 & 8) != 0, w8, 1.0)
    v = v[None]

    m = jnp.maximum(pos_ref[...], 1)
    sh = jnp.right_shift(m[:, :, None], t)
    bit = (sh >= 2) & ((sh & 1) == par)
    out_ref[...] = jnp.where(bit, v, 0.0)


@functools.partial(jax.jit, static_argnames=("block_b", "word_len"))
def _expand(positions, weight_row, block_b=64, word_len=20):
    b, s = positions.shape
    d = weight_row.shape[1]
    c = word_len * d
    grid = (b // block_b,)
    return pl.pallas_call(
        _expand_body,
        grid=grid,
        in_specs=[
            pl.BlockSpec((block_b, s), lambda i: (i, 0)),
            pl.BlockSpec((1, d), lambda i: (0, 0)),
        ],
        out_specs=pl.BlockSpec((block_b, s, c), lambda i: (i, 0, 0)),
        out_shape=jax.ShapeDtypeStruct((b, s, c), jnp.float32),
        compiler_params=pltpu.CompilerParams(
            dimension_semantics=("arbitrary",),
        ),
    )(positions, weight_row)


def kernel(positions, weight, paths):
    d = weight.shape[0]
    word_len = paths.shape[1]
    return _expand(positions, weight.reshape(1, d), word_len=word_len)
